# Initial kernel scaffold; baseline (speedup 1.0000x reference)
import jax
import jax.numpy as jnp
from jax.experimental import pallas as pl


def kernel(inputs, embedding):
    raise NotImplementedError("write your pallas kernel here")


# speedup vs baseline: 1.8131x; 1.8131x over previous
import jax
import jax.numpy as jnp
from jax.experimental import pallas as pl

NUM_CODES = 1024
DIM = 64


def _vq_block(x_ref, e_ref, o_ref):
    x = x_ref[...]
    e = e_ref[...]
    ee = e * e
    ones = jnp.ones((8, DIM), jnp.float32)
    en = jax.lax.dot_general(
        ones, ee, (((1,), (1,)), ((), ())), preferred_element_type=jnp.float32)
    mm = jax.lax.dot_general(
        x, e, (((1,), (1,)), ((), ())), preferred_element_type=jnp.float32)
    d = en[0:1, :] - 2.0 * mm
    m = d[:, 0:128]
    for k in range(1, 8):
        m = jnp.minimum(m, d[:, 128 * k:128 * (k + 1)])
    m = jnp.min(m, axis=1, keepdims=True)
    onehot = (d <= m).astype(jnp.float32)
    closest = jax.lax.dot_general(
        onehot, e, (((1,), (0,)), ((), ())), preferred_element_type=jnp.float32)
    o_ref[...] = closest + (x - closest)


def kernel(inputs, embedding):
    orig_shape = inputs.shape
    flat = inputs.reshape(-1, DIM)
    n = flat.shape[0]
    block = 1152
    grid = (n // block,)
    out = pl.pallas_call(
        _vq_block,
        grid=grid,
        in_specs=[
            pl.BlockSpec((block, DIM), lambda i: (i, 0)),
            pl.BlockSpec((NUM_CODES, DIM), lambda i: (0, 0)),
        ],
        out_specs=pl.BlockSpec((block, DIM), lambda i: (i, 0)),
        out_shape=jax.ShapeDtypeStruct((n, DIM), jnp.float32),
    )(flat, embedding)
    return out.reshape(orig_shape)
